# baseline (speedup 1.0000x reference)
import jax
import jax.numpy as jnp
from jax.experimental import pallas as pl
from jax.experimental.pallas import tpu as pltpu

K = 100000
KB = 25000
NB = K // KB
NQ = 16
D = 64


def _nn_body(mem_ref, q_ref, matched_ref, minv_ref, bestv_scr):
    i = pl.program_id(0)
    mem = mem_ref[...]
    q = q_ref[...]
    ones = jnp.ones((1, D), dtype=jnp.float32)
    msq = mem * mem
    norms = jax.lax.dot_general(
        ones, msq, (((1,), (1,)), ((), ())),
        preferred_element_type=jnp.float32,
        precision=jax.lax.Precision.HIGHEST)
    dots = jax.lax.dot_general(
        q, mem, (((1,), (1,)), ((), ())),
        preferred_element_type=jnp.float32,
        precision=jax.lax.Precision.HIGHEST)
    dist = norms - 2.0 * dots
    bmin = jnp.min(dist, axis=1, keepdims=True)
    cols = jax.lax.broadcasted_iota(jnp.int32, (NQ, KB), 1)
    onehot = jnp.where(dist == bmin, jnp.float32(1.0), jnp.float32(0.0))
    bcol = jnp.min(jnp.where(dist == bmin, cols, K), axis=1, keepdims=True)
    onehot = jnp.where(cols == bcol, onehot, jnp.float32(0.0))
    rowsel = jax.lax.dot_general(
        onehot, mem, (((1,), (0,)), ((), ())),
        preferred_element_type=jnp.float32)

    @pl.when(i == 0)
    def _init():
        bestv_scr[...] = bmin
        matched_ref[...] = rowsel

    @pl.when(i > 0)
    def _update():
        prev = bestv_scr[...]
        upd = bmin < prev
        bestv_scr[...] = jnp.where(upd, bmin, prev)
        matched_ref[...] = jnp.where(
            jnp.broadcast_to(upd, (NQ, D)), rowsel, matched_ref[...])

    @pl.when(i == NB - 1)
    def _final():
        qn = jnp.sum(q * q, axis=1, keepdims=True)
        minv_ref[...] = jnp.min(bestv_scr[...] + qn).reshape(1, 1)


def kernel(inpt, in_memory):
    matched, minv = pl.pallas_call(
        _nn_body,
        grid=(NB,),
        in_specs=[
            pl.BlockSpec((KB, D), lambda i: (i, 0)),
            pl.BlockSpec((NQ, D), lambda i: (0, 0)),
        ],
        out_specs=[
            pl.BlockSpec((NQ, D), lambda i: (0, 0)),
            pl.BlockSpec((1, 1), lambda i: (0, 0)),
        ],
        out_shape=[
            jax.ShapeDtypeStruct((NQ, D), jnp.float32),
            jax.ShapeDtypeStruct((1, 1), jnp.float32),
        ],
        scratch_shapes=[pltpu.VMEM((NQ, 1), jnp.float32)],
        compiler_params=pltpu.CompilerParams(
            dimension_semantics=("arbitrary",)),
    )(in_memory, inpt)
    return matched, minv[0, 0]


# speedup vs baseline: 1.0111x; 1.0111x over previous
import jax
import jax.numpy as jnp
from jax.experimental import pallas as pl
from jax.experimental.pallas import tpu as pltpu

K = 100000
KB = 20000
NB = K // KB
NQ = 16
D = 64


def _nn_body(mem_ref, q_ref, matched_ref, minv_ref, bestv_scr):
    i = pl.program_id(0)
    mem = mem_ref[...]
    q = q_ref[...]
    ones = jnp.ones((1, D), dtype=jnp.float32)
    msq = mem * mem
    norms = jax.lax.dot_general(
        ones, msq, (((1,), (1,)), ((), ())),
        preferred_element_type=jnp.float32,
        precision=jax.lax.Precision.HIGHEST)
    dots = jax.lax.dot_general(
        q, mem, (((1,), (1,)), ((), ())),
        preferred_element_type=jnp.float32,
        precision=jax.lax.Precision.HIGHEST)
    dist = norms - 2.0 * dots
    bmin = jnp.min(dist, axis=1, keepdims=True)
    cols = jax.lax.broadcasted_iota(jnp.int32, (NQ, KB), 1)
    onehot = jnp.where(dist == bmin, jnp.float32(1.0), jnp.float32(0.0))
    bcol = jnp.min(jnp.where(dist == bmin, cols, K), axis=1, keepdims=True)
    onehot = jnp.where(cols == bcol, onehot, jnp.float32(0.0))
    rowsel = jax.lax.dot_general(
        onehot, mem, (((1,), (0,)), ((), ())),
        preferred_element_type=jnp.float32)

    @pl.when(i == 0)
    def _init():
        bestv_scr[...] = bmin
        matched_ref[...] = rowsel

    @pl.when(i > 0)
    def _update():
        prev = bestv_scr[...]
        upd = bmin < prev
        bestv_scr[...] = jnp.where(upd, bmin, prev)
        matched_ref[...] = jnp.where(
            jnp.broadcast_to(upd, (NQ, D)), rowsel, matched_ref[...])

    @pl.when(i == NB - 1)
    def _final():
        qn = jnp.sum(q * q, axis=1, keepdims=True)
        minv_ref[...] = jnp.min(bestv_scr[...] + qn).reshape(1, 1)


def kernel(inpt, in_memory):
    matched, minv = pl.pallas_call(
        _nn_body,
        grid=(NB,),
        in_specs=[
            pl.BlockSpec((KB, D), lambda i: (i, 0)),
            pl.BlockSpec((NQ, D), lambda i: (0, 0)),
        ],
        out_specs=[
            pl.BlockSpec((NQ, D), lambda i: (0, 0)),
            pl.BlockSpec((1, 1), lambda i: (0, 0)),
        ],
        out_shape=[
            jax.ShapeDtypeStruct((NQ, D), jnp.float32),
            jax.ShapeDtypeStruct((1, 1), jnp.float32),
        ],
        scratch_shapes=[pltpu.VMEM((NQ, 1), jnp.float32)],
        compiler_params=pltpu.CompilerParams(
            dimension_semantics=("arbitrary",)),
    )(in_memory, inpt)
    return matched, minv[0, 0]
